# baseline (speedup 1.0000x reference)
import functools

import jax
import jax.numpy as jnp
from jax import lax
from jax.experimental import pallas as pl
from jax.experimental.pallas import tpu as pltpu
from jax.experimental.pallas import tpu_sc as plsc

VOCAB = 1000
VOCAB_PAD = 1024
EMBED_DIM = 128
LP = 24
NB = 2


def _table_kernel(e_ref, w_ref, bias_ref, out_ref):
    acc = lax.dot_general(
        e_ref[...],
        w_ref[...],
        dimension_numbers=(((1,), (1,)), ((), ())),
        preferred_element_type=jnp.float32,
        precision=lax.Precision.HIGHEST,
    )
    out_ref[...] = acc + bias_ref[...]


def _make_table(embed_table, W, b):
    w_pad = jnp.zeros((VOCAB_PAD, EMBED_DIM), jnp.float32).at[:VOCAB].set(W)
    b_pad = jnp.zeros((1, VOCAB_PAD), jnp.float32).at[0, :VOCAB].set(b)
    return pl.pallas_call(
        _table_kernel,
        out_shape=jax.ShapeDtypeStruct((VOCAB, VOCAB_PAD), jnp.float32),
    )(embed_table, w_pad, b_pad)


def _gather_fn(B, L):
    info = plsc.get_sparse_core_info()
    nc, ns = info.num_cores, info.num_subcores
    nw = nc * ns
    assert B % (nw * NB) == 0
    nbat = B // (nw * NB)
    nidx = NB * LP
    mesh = plsc.VectorSubcoreMesh(core_axis_name="c", subcore_axis_name="s")

    @functools.partial(
        pl.kernel,
        mesh=mesh,
        out_type=jax.ShapeDtypeStruct((B * LP, VOCAB_PAD), jnp.float32),
        scratch_types=[
            pltpu.VMEM((nbat, nidx), jnp.int32),
            pltpu.VMEM((nidx, VOCAB_PAD), jnp.float32),
            pltpu.VMEM((nidx, VOCAB_PAD), jnp.float32),
            pltpu.SemaphoreType.DMA,
            pltpu.SemaphoreType.DMA,
            pltpu.SemaphoreType.DMA,
            pltpu.SemaphoreType.DMA,
        ],
    )
    def gather(idx_hbm, table_hbm, out_hbm,
               idx_v, rows0, rows1, semg0, semg1, sems0, sems1):
        wid = lax.axis_index("s") * nc + lax.axis_index("c")
        b0 = wid * (nbat * NB)

        pltpu.sync_copy(idx_hbm.at[wid], idx_v)

        def fire_gather(c, rows, sem):
            pltpu.async_copy(table_hbm.at[idx_v.at[c]], rows, sem)

        def wait_gather(c, rows, sem):
            pltpu.make_async_copy(table_hbm.at[idx_v.at[c]], rows, sem).wait()

        def fire_stores(c, rows, sem):
            pltpu.async_copy(
                rows, out_hbm.at[pl.ds((b0 + NB * c) * LP, NB * LP)], sem)

        def wait_stores(c, rows, sem):
            pltpu.make_async_copy(
                rows, out_hbm.at[pl.ds((b0 + NB * c) * LP, NB * LP)],
                sem).wait()

        bufs = ((rows0, semg0, sems0), (rows1, semg1, sems1))
        fire_gather(0, rows0, semg0)

        def body(c, carry):
            even = lax.rem(c, 2) == 0

            def step(p):
                rows_p, semg_p, sems_p = bufs[p]
                rows_q, semg_q, sems_q = bufs[1 - p]

                @pl.when(c >= 1)
                def _():
                    wait_stores(c - 1, rows_q, sems_q)

                @pl.when(c + 1 < nbat)
                def _():
                    fire_gather(c + 1, rows_q, semg_q)

                wait_gather(c, rows_p, semg_p)
                fire_stores(c, rows_p, sems_p)

            @pl.when(even)
            def _():
                step(0)

            @pl.when(jnp.logical_not(even))
            def _():
                step(1)

            return carry

        lax.fori_loop(0, nbat, body, 0)
        last = nbat - 1
        rows_l, _, sems_l = bufs[last % 2]
        wait_stores(last, rows_l, sems_l)

    return gather


def kernel(x, embed_table, W, b):
    B, L = x.shape
    table = _make_table(embed_table, W, b)
    info = plsc.get_sparse_core_info()
    nw = info.num_cores * info.num_subcores
    xp = jnp.pad(x.astype(jnp.int32), ((0, 0), (0, LP - L)))
    idx = xp.reshape(nw, B // (nw * NB), NB * LP)
    out2d = _gather_fn(B, L)(idx, table)
    return out2d.reshape(B, LP, VOCAB_PAD)[:, :L, :VOCAB]


# speedup vs baseline: 1.8711x; 1.8711x over previous
import functools

import jax
import jax.numpy as jnp
from jax import lax
from jax.experimental import pallas as pl
from jax.experimental.pallas import tpu as pltpu
from jax.experimental.pallas import tpu_sc as plsc

VOCAB = 1000
VOCAB_PAD = 1024
EMBED_DIM = 128
CHUNK = 32


def _table_kernel(e_ref, w_ref, bias_ref, out_ref):
    acc = lax.dot_general(
        e_ref[...],
        w_ref[...],
        dimension_numbers=(((1,), (1,)), ((), ())),
        preferred_element_type=jnp.float32,
        precision=lax.Precision.HIGHEST,
    )
    out_ref[...] = acc + bias_ref[...]


def _make_table(embed_table, W, b):
    w_pad = jnp.zeros((VOCAB_PAD, EMBED_DIM), jnp.float32).at[:VOCAB].set(W)
    b_pad = jnp.zeros((1, VOCAB_PAD), jnp.float32).at[0, :VOCAB].set(b)
    return pl.pallas_call(
        _table_kernel,
        out_shape=jax.ShapeDtypeStruct((VOCAB, VOCAB_PAD), jnp.float32),
    )(embed_table, w_pad, b_pad)


def _gather_fn(n_tokens):
    info = plsc.get_sparse_core_info()
    nc, ns = info.num_cores, info.num_subcores
    nw = nc * ns
    assert n_tokens % (nw * CHUNK) == 0
    nchunk = n_tokens // (nw * CHUNK)
    mesh = plsc.VectorSubcoreMesh(core_axis_name="c", subcore_axis_name="s")

    @functools.partial(
        pl.kernel,
        mesh=mesh,
        out_type=jax.ShapeDtypeStruct((n_tokens, VOCAB_PAD), jnp.float32),
        scratch_types=[
            pltpu.VMEM((nchunk, CHUNK), jnp.int32),
            pltpu.VMEM((CHUNK, VOCAB_PAD), jnp.float32),
            pltpu.VMEM((CHUNK, VOCAB_PAD), jnp.float32),
            pltpu.SemaphoreType.DMA,
            pltpu.SemaphoreType.DMA,
            pltpu.SemaphoreType.DMA,
            pltpu.SemaphoreType.DMA,
        ],
    )
    def gather(idx_hbm, table_hbm, out_hbm,
               idx_v, rows0, rows1, semg0, semg1, sems0, sems1):
        wid = lax.axis_index("s") * nc + lax.axis_index("c")
        base = wid * (nchunk * CHUNK)
        pltpu.sync_copy(idx_hbm.at[wid], idx_v)

        def fire_gather(c, rows, sem):
            pltpu.async_copy(table_hbm.at[idx_v.at[c]], rows, sem)

        def wait_gather(c, rows, sem):
            pltpu.make_async_copy(table_hbm.at[idx_v.at[c]], rows, sem).wait()

        def fire_stores(c, rows, sem):
            pltpu.async_copy(rows, out_hbm.at[pl.ds(base + c * CHUNK, CHUNK)],
                             sem)

        def wait_stores(c, rows, sem):
            pltpu.make_async_copy(
                rows, out_hbm.at[pl.ds(base + c * CHUNK, CHUNK)], sem).wait()

        bufs = ((rows0, semg0, sems0), (rows1, semg1, sems1))
        fire_gather(0, rows0, semg0)

        def body(c, carry):
            even = lax.rem(c, 2) == 0

            def step(p):
                rows_p, semg_p, sems_p = bufs[p]
                rows_q, semg_q, sems_q = bufs[1 - p]

                @pl.when(c >= 1)
                def _():
                    wait_stores(c - 1, rows_q, sems_q)

                @pl.when(c + 1 < nchunk)
                def _():
                    fire_gather(c + 1, rows_q, semg_q)

                wait_gather(c, rows_p, semg_p)
                fire_stores(c, rows_p, sems_p)

            @pl.when(even)
            def _():
                step(0)

            @pl.when(jnp.logical_not(even))
            def _():
                step(1)

            return carry

        lax.fori_loop(0, nchunk, body, 0)
        last = nchunk - 1
        rows_l, _, sems_l = bufs[last % 2]
        wait_stores(last, rows_l, sems_l)

    return gather


def kernel(x, embed_table, W, b):
    B, L = x.shape
    n_tokens = B * L
    table = _make_table(embed_table, W, b)
    info = plsc.get_sparse_core_info()
    nw = info.num_cores * info.num_subcores
    idx = x.reshape(nw, n_tokens // (nw * CHUNK), CHUNK).astype(jnp.int32)
    out = _gather_fn(n_tokens)(idx, table)
    return out[:, :VOCAB].reshape(B, L, VOCAB)
